# Initial kernel scaffold; baseline (speedup 1.0000x reference)
import jax
import jax.numpy as jnp
from jax.experimental import pallas as pl


def kernel(ids, weight):
    raise NotImplementedError("write your pallas kernel here")


# speedup vs baseline: 2.9636x; 2.9636x over previous
import functools

import jax
import jax.numpy as jnp
from jax import lax
from jax.experimental import pallas as pl
from jax.experimental.pallas import tpu as pltpu
from jax.experimental.pallas import tpu_sc as plsc

VOCAB = 100000
HIDDEN = 128
BATCH = 4096
HIST = 50

NC = 2
NS = 16
NW = NC * NS
ROWS = BATCH * HIST
BPW = ROWS // NW
CH = 128
NCH = BPW // CH


def _gather_kernel(idx_hbm, w_hbm, out_hbm, idx_v, buf, sem):
    wid = lax.axis_index("s") * NC + lax.axis_index("c")
    base = wid * BPW
    pltpu.sync_copy(idx_hbm.at[wid], idx_v)

    def body(c, carry):
        pltpu.async_copy(w_hbm.at[idx_v.at[c]], buf, sem).wait()
        pltpu.sync_copy(buf, out_hbm.at[pl.ds(base + c * CH, CH)])
        return carry

    lax.fori_loop(0, NCH, body, 0)


def kernel(ids, weight):
    idx = ids.astype(jnp.int32).reshape(NW, NCH, CH)
    mesh = plsc.VectorSubcoreMesh(
        core_axis_name="c", subcore_axis_name="s", num_cores=NC, num_subcores=NS
    )
    run = pl.kernel(
        _gather_kernel,
        out_type=jax.ShapeDtypeStruct((ROWS, HIDDEN), jnp.float32),
        mesh=mesh,
        scratch_types=[
            pltpu.VMEM((NCH, CH), jnp.int32),
            pltpu.VMEM((CH, HIDDEN), jnp.float32),
            pltpu.SemaphoreType.DMA,
        ],
    )
    out = run(idx, weight)
    return out.reshape(BATCH, HIST, HIDDEN)


# speedup vs baseline: 3.3463x; 1.1292x over previous
import jax
import jax.numpy as jnp
from jax import lax
from jax.experimental import pallas as pl
from jax.experimental.pallas import tpu as pltpu
from jax.experimental.pallas import tpu_sc as plsc

VOCAB = 100000
HIDDEN = 128
BATCH = 4096
HIST = 50

NC = 2
NS = 16
NW = NC * NS
ROWS = BATCH * HIST
BPW = ROWS // NW
CH = 128
NCH = BPW // CH
NBUF = 5


def _gather_kernel(idx_hbm, w_hbm, out_hbm, idx_v,
                   b0, b1, b2, b3, b4,
                   g0, g1, g2, g3, g4,
                   w0, w1, w2, w3, w4):
    bufs = (b0, b1, b2, b3, b4)
    gsems = (g0, g1, g2, g3, g4)
    wsems = (w0, w1, w2, w3, w4)
    wid = lax.axis_index("s") * NC + lax.axis_index("c")
    base = wid * BPW
    pltpu.sync_copy(idx_hbm.at[wid], idx_v)

    def g_desc(c, b):
        return pltpu.make_async_copy(w_hbm.at[idx_v.at[c]], bufs[b], gsems[b])

    def w_desc(c, b):
        return pltpu.make_async_copy(
            bufs[b], out_hbm.at[pl.ds(base + c * CH, CH)], wsems[b])

    def step(c, b):
        g_desc(c, b).wait()
        w_desc(c, b).start()
        bp = (b - 1) % NBUF
        w_desc(c - 1, bp).wait()
        g_desc(c + NBUF - 1, bp).start()

    for b in range(NBUF):
        g_desc(b, b).start()

    g_desc(0, 0).wait()
    w_desc(0, 0).start()
    for c in range(1, NBUF):
        step(c, c % NBUF)

    def outer(k, carry):
        c0 = k * NBUF
        for b in range(NBUF):
            step(c0 + b, b)
        return carry

    lax.fori_loop(1, (NCH - NBUF) // NBUF, outer, 0)

    for c in range(NCH - NBUF, NCH):
        b = c % NBUF
        g_desc(c, b).wait()
        w_desc(c, b).start()
        bp = (b - 1) % NBUF
        w_desc(c - 1, bp).wait()
        if c + NBUF - 1 < NCH:
            g_desc(c + NBUF - 1, bp).start()
    w_desc(NCH - 1, (NCH - 1) % NBUF).wait()


def kernel(ids, weight):
    idx = ids.astype(jnp.int32).reshape(NW, NCH, CH)
    mesh = plsc.VectorSubcoreMesh(
        core_axis_name="c", subcore_axis_name="s", num_cores=NC, num_subcores=NS
    )
    run = pl.kernel(
        _gather_kernel,
        out_type=jax.ShapeDtypeStruct((ROWS, HIDDEN), jnp.float32),
        mesh=mesh,
        scratch_types=(
            [pltpu.VMEM((NCH, CH), jnp.int32)]
            + [pltpu.VMEM((CH, HIDDEN), jnp.float32) for _ in range(NBUF)]
            + [pltpu.SemaphoreType.DMA for _ in range(2 * NBUF)]
        ),
    )
    out = run(idx, weight)
    return out.reshape(BATCH, HIST, HIDDEN)
